# Initial kernel scaffold; baseline (speedup 1.0000x reference)
import jax
import jax.numpy as jnp
from jax.experimental import pallas as pl


def kernel(x, edge_index, edge_attr, W1p, R1p, b1p, W1e, R1e, b1e, W2p, R2p, b2p, W2e, R2e, b2e, L1, c1l, L2, c2l):
    raise NotImplementedError("write your pallas kernel here")


# speedup vs baseline: 20.0883x; 20.0883x over previous
import functools

import jax
import jax.numpy as jnp
from jax import lax
from jax.experimental import pallas as pl
from jax.experimental.pallas import tpu as pltpu
from jax.experimental.pallas import tpu_sc as plsc

NC, NS = 2, 16
NW = NC * NS
CH = 128


def _sc_edge_pass(NP, CPT, D, scaled):
    RPS = NP // NS
    mesh = plsc.VectorSubcoreMesh(core_axis_name="c", subcore_axis_name="s")
    scratch = [
        pltpu.VMEM((CPT, CH), jnp.int32),
        pltpu.VMEM((CPT, CH), jnp.int32),
        pltpu.VMEM((CH, D), jnp.float32),
        pltpu.VMEM((RPS, D), jnp.float32),
        pltpu.VMEM_SHARED((NP, D), jnp.float32),
        pltpu.SemaphoreType.DMA,
    ]
    if scaled:
        scratch.insert(2, pltpu.VMEM((CPT, CH), jnp.float32))

    def body(*refs):
        if scaled:
            (table, g2d, s2d, ea2d, zeros, out,
             gv, sv, eav, vals, zbuf, acc, sem) = refs
        else:
            (table, g2d, s2d, zeros, out,
             gv, sv, vals, zbuf, acc, sem) = refs
        c = lax.axis_index("c")
        s = lax.axis_index("s")
        w = c * NS + s
        pltpu.sync_copy(zeros.at[pl.ds(s * RPS, RPS), :], zbuf)
        pltpu.sync_copy(zbuf, acc.at[pl.ds(s * RPS, RPS), :])
        pltpu.sync_copy(g2d.at[pl.ds(w * CPT, CPT), :], gv)
        pltpu.sync_copy(s2d.at[pl.ds(w * CPT, CPT), :], sv)
        if scaled:
            pltpu.sync_copy(ea2d.at[pl.ds(w * CPT, CPT), :], eav)
        plsc.subcore_barrier()

        def chunk(j, carry):
            pltpu.async_copy(table.at[gv.at[j]], vals, sem).wait()
            if scaled:
                for k in range(CH // 16):
                    ev = eav[j, pl.ds(k * 16, 16)]
                    for t in range(16):
                        i = k * 16 + t
                        vals[i, :] = vals[i, :] * ev[t]
            pltpu.sync_copy(vals, acc.at[sv.at[j]], add=True)
            return carry

        lax.fori_loop(0, CPT, chunk, 0)
        plsc.subcore_barrier()
        pltpu.sync_copy(acc.at[pl.ds(s * RPS, RPS), :],
                        out.at[c, pl.ds(s * RPS, RPS), :])

    return pl.kernel(
        body,
        out_type=jax.ShapeDtypeStruct((NC, NP, D), jnp.float32),
        mesh=mesh,
        scratch_types=scratch,
        compiler_params=pltpu.CompilerParams(use_tc_tiling_on_sc=False),
    )


def _tc_node_body(N, part, xp, W1p, R1p, b1p, W1e, R1e, b1e, ssm_o, x1_o):
    agg = part[0] + part[1]
    cnt = agg[:, 5:6]
    mean5 = agg[:, 0:5] / jnp.maximum(cnt, 1.0)
    xr = xp[:, 0:5]
    NPl = xr.shape[0]
    s = mean5 @ W1p[...] + xr @ R1p[...] + b1p[...]
    x1 = jnp.maximum(mean5 @ W1e[...] + xr @ R1e[...] + b1e[...], 0.0)
    m = jnp.max(s, axis=-1, keepdims=True)
    es = jnp.exp(s - m)
    sm = es / jnp.sum(es, axis=-1, keepdims=True)
    valid = lax.broadcasted_iota(jnp.int32, (NPl, 1), 0) < N
    ssm_o[...] = jnp.where(valid, sm, 0.0)
    x1_o[...] = jnp.where(valid, x1, 0.0)


def _tc_head_body(part, ssm, x1, W2e, R2e, b2e, L1, c1l, L2, c2l,
                  logp_o, reg_o):
    y = part[0] + part[1]
    cd = (((0,), (0,)), ((), ()))
    adj = lax.dot_general(ssm[...], y, cd,
                          preferred_element_type=jnp.float32)
    out1 = lax.dot_general(ssm[...], x1[...], cd,
                           preferred_element_type=jnp.float32)
    ii = lax.broadcasted_iota(jnp.int32, (16, 16), 0)
    jj = lax.broadcasted_iota(jnp.int32, (16, 16), 1)
    eye = ii == jj
    reg1 = -jnp.sum(jnp.where(eye, adj, 0.0))
    mean0 = jnp.mean(out1, axis=0, keepdims=True)
    x2 = jnp.maximum(mean0 @ W2e[...] + out1 @ R2e[...] + b2e[...], 0.0)
    out2 = jnp.sum(x2, axis=0, keepdims=True)
    de = jnp.where(eye, 0.0, adj)
    row_sum = jnp.sum(de, axis=1, keepdims=True)
    de2 = jnp.where(eye, jnp.broadcast_to(row_sum, (16, 16)), -de)
    reg2 = jnp.sum(de2)
    h = jnp.maximum(out2 @ L1[...] + c1l[...], 0.0)
    h2 = h @ L2[...] + c2l[...]
    mm = jnp.max(h2, axis=1, keepdims=True)
    lse = mm + jnp.log(jnp.sum(jnp.exp(h2 - mm), axis=1, keepdims=True))
    logp_o[...] = h2 - lse
    reg_o[...] = jnp.reshape(reg1 + reg2, (1, 1))


def kernel(x, edge_index, edge_attr, W1p, R1p, b1p, W1e, R1e, b1e,
           W2p, R2p, b2p, W2e, R2e, b2e, L1, c1l, L2, c2l):
    N = x.shape[0]
    E = edge_index.shape[1]
    NP = ((N + 1 + 127) // 128) * 128
    CPT = ((-(-E // (NW * CH)) + 7) // 8) * 8
    EP = CPT * CH * NW

    row = edge_index[0].astype(jnp.int32)
    col = edge_index[1].astype(jnp.int32)
    ea = edge_attr.reshape(-1).astype(jnp.float32)
    pad_e = EP - E
    rowp = jnp.concatenate([row, jnp.full((pad_e,), N, jnp.int32)])
    colp = jnp.concatenate([col, jnp.full((pad_e,), N, jnp.int32)])
    eap = jnp.concatenate([ea, jnp.zeros((pad_e,), jnp.float32)])
    row2d = rowp.reshape(EP // CH, CH)
    col2d = colp.reshape(EP // CH, CH)
    ea2d = eap.reshape(EP // CH, CH)

    xp = jnp.concatenate(
        [x, jnp.ones((N, 1), jnp.float32), jnp.zeros((N, 2), jnp.float32)],
        axis=1)
    xp = jnp.pad(xp, ((0, NP - N), (0, 0)))
    zeros8 = jnp.zeros((NP, 8), jnp.float32)
    zeros16 = jnp.zeros((NP, 16), jnp.float32)

    part_a = _sc_edge_pass(NP, CPT, 8, scaled=False)(
        xp, row2d, col2d, zeros8)

    node = pl.pallas_call(
        functools.partial(_tc_node_body, N),
        out_shape=[
            jax.ShapeDtypeStruct((NP, 16), jnp.float32),
            jax.ShapeDtypeStruct((NP, 32), jnp.float32),
        ],
    )
    ssm, x1 = node(part_a, xp, W1p, R1p, b1p.reshape(1, 16),
                   W1e, R1e, b1e.reshape(1, 32))

    part_c = _sc_edge_pass(NP, CPT, 16, scaled=True)(
        ssm, col2d, row2d, ea2d, zeros16)

    head = pl.pallas_call(
        _tc_head_body,
        out_shape=[
            jax.ShapeDtypeStruct((1, 2), jnp.float32),
            jax.ShapeDtypeStruct((1, 1), jnp.float32),
        ],
    )
    logp, reg = head(part_c, ssm, x1, W2e, R2e, b2e.reshape(1, 16),
                     L1, c1l.reshape(1, 8), L2, c2l.reshape(1, 2))
    return logp[0], reg[0, 0]
